# baseline (speedup 1.0000x reference)
import functools

import jax
import jax.numpy as jnp
from jax.experimental import pallas as pl

MAX_N = 8
NVT = 26
MAX_POS = 9
HS = 301
HSP = 384
EMB = 16
FEAT_EMB = 8
NZ = 56
XDIM = NVT + MAX_POS
B = 4096
BB = 512


def _sig(x):
    return 0.5 * jnp.tanh(0.5 * x) + 0.5


def _dot(a, b):
    return jax.lax.dot_general(a.astype(jnp.bfloat16), b, (((1,), (0,)), ((), ())),
                               preferred_element_type=jnp.float32)


def _gru(gi, gh, hin):
    r = _sig(gi[:, 0:HSP] + gh[:, 0:HSP])
    z = _sig(gi[:, HSP:2 * HSP] + gh[:, HSP:2 * HSP])
    n = jnp.tanh(gi[:, 2 * HSP:3 * HSP] + r * gh[:, 2 * HSP:3 * HSP])
    return (1.0 - z) * n + z * hin


def _kernel_body(vt_ref, vp_ref, adj_ref, feats_ref,
                 wih_ref, whh_ref, bih_ref, bhh_ref,
                 wgm_ref, wgmp_ref, bgm_ref,
                 dfw1_ref, dfb1_ref, dfw2_ref, dfb2_ref,
                 fc1h_ref, fc1d_ref, fc1b_ref,
                 fc2h_ref, fc2d_ref, fc2b_ref,
                 mu_ref, lv_ref):
    vt = vt_ref[...]
    vp = vp_ref[...]
    adjf = adj_ref[...].astype(jnp.float32)
    bih = bih_ref[...]
    bhh = bhh_ref[...]
    bgm = bgm_ref[...]

    iota_x = jax.lax.broadcasted_iota(jnp.int32, (BB, XDIM), 1)
    iota_p = jax.lax.broadcasted_iota(jnp.int32, (BB, MAX_POS), 1)

    def onehot_x(v):
        t = vt[:, v:v + 1]
        p = vp[:, v:v + 1] + NVT
        return ((iota_x == t) | (iota_x == p)).astype(jnp.float32)

    def onehot_p(v):
        return (iota_p == vp[:, v:v + 1]).astype(jnp.float32)

    def gi_of(v):
        return _dot(onehot_x(v), wih_ref[...]) + bih

    h = _gru(gi_of(0), jnp.broadcast_to(bhh, (BB, 3 * HSP)), jnp.zeros((BB, HSP), jnp.float32))

    gated = []
    for v in range(1, MAX_N):
        u = v - 1
        gm = _dot(h, wgm_ref[...]) + _dot(onehot_p(u), wgmp_ref[...]) + bgm
        gated.append(_sig(gm[:, 0:HSP]) * gm[:, HSP:2 * HSP])
        hagg = jnp.zeros((BB, HSP), jnp.float32)
        for u2 in range(v):
            m = adjf[:, u2 * MAX_N + v:u2 * MAX_N + v + 1]
            hagg = hagg + m * gated[u2]
        gh = _dot(hagg, whh_ref[...]) + bhh
        h = _gru(gi_of(v), gh, hagg)

    iota_d = jax.lax.broadcasted_iota(jnp.int32, (BB, 3 * MAX_POS), 1) // 3
    df = jnp.zeros((BB, 3 * MAX_POS), jnp.float32)
    for v in range(MAX_N):
        newv = feats_ref[:, 27 * v:27 * (v + 1)]
        df = jnp.where(iota_d == vp[:, v:v + 1], newv, df)

    hd1 = jnp.maximum(_dot(df, dfw1_ref[...]) + dfb1_ref[...], 0.0)
    hd = _dot(hd1, dfw2_ref[...]) + dfb2_ref[...]

    mu_ref[...] = _dot(h, fc1h_ref[...]) + _dot(hd, fc1d_ref[...]) + fc1b_ref[...]
    lv_ref[...] = _dot(h, fc2h_ref[...]) + _dot(hd, fc2d_ref[...]) + fc2b_ref[...]


def _pad_rows(w, rows):
    return jnp.pad(w, ((0, rows - w.shape[0]), (0, 0)))


def _pad3(wT):
    k = wT.shape[0]
    out = jnp.zeros((k, 3 * HSP), wT.dtype)
    for c in range(3):
        out = out.at[:, c * HSP:c * HSP + HS].set(wT[:, c * HS:(c + 1) * HS])
    return out


@jax.jit
def kernel(v_types, v_pos, adj, feats, W_ih, W_hh, b_ih, b_hh, gate_W, gate_b,
           map_W, df_W1, df_b1, df_W2, df_b2, fc1_W, fc1_b, fc2_W, fc2_b):
    f32 = jnp.float32
    wih = _pad3(W_ih.T)
    whh = _pad_rows(_pad3(W_hh.T), HSP)
    bih = _pad3(b_ih[None, :])
    bhh = _pad3(b_hh[None, :])
    gW_h, gW_p = gate_W[:, :HS], gate_W[:, HS:]
    mW_h, mW_p = map_W[:, :HS], map_W[:, HS:]
    pad_h = lambda w: _pad_rows(jnp.pad(w.T, ((0, 0), (0, HSP - HS))), HSP)
    wgm = jnp.concatenate([pad_h(gW_h), pad_h(mW_h)], axis=1)
    pad_p = lambda w: jnp.pad(w.T, ((0, 0), (0, HSP - HS)))
    wgmp = jnp.concatenate([pad_p(gW_p), pad_p(mW_p)], axis=1)
    bgm = jnp.pad(gate_b[None, :], ((0, 0), (0, 2 * HSP - HS)))

    dfw1 = df_W1.T
    dfb1 = df_b1[None, :]
    dfw2 = df_W2.T
    dfb2 = df_b2[None, :]
    fc1h = _pad_rows(fc1_W[:, :HS].T, HSP)
    fc1d = fc1_W[:, HS:].T
    fc1b = fc1_b[None, :]
    fc2h = _pad_rows(fc2_W[:, :HS].T, HSP)
    fc2d = fc2_W[:, HS:].T
    fc2b = fc2_b[None, :]

    adj2 = adj.reshape(B, MAX_N * MAX_N)
    feats2 = jnp.tile(feats, (1, 1, MAX_POS)).reshape(B, MAX_N * 3 * MAX_POS)

    nb = B // BB
    data_spec = lambda cols: pl.BlockSpec((BB, cols), lambda i: (i, 0))
    w_spec = lambda r, c: pl.BlockSpec((r, c), lambda i: (0, 0))

    in_specs = [
        data_spec(MAX_N), data_spec(MAX_N), data_spec(MAX_N * MAX_N),
        data_spec(MAX_N * 3 * MAX_POS),
        w_spec(XDIM, 3 * HSP), w_spec(HSP, 3 * HSP),
        w_spec(1, 3 * HSP), w_spec(1, 3 * HSP),
        w_spec(HSP, 2 * HSP), w_spec(MAX_POS, 2 * HSP), w_spec(1, 2 * HSP),
        w_spec(3 * MAX_POS, EMB), w_spec(1, EMB),
        w_spec(EMB, FEAT_EMB), w_spec(1, FEAT_EMB),
        w_spec(HSP, NZ), w_spec(FEAT_EMB, NZ), w_spec(1, NZ),
        w_spec(HSP, NZ), w_spec(FEAT_EMB, NZ), w_spec(1, NZ),
    ]
    out_specs = [data_spec(NZ), data_spec(NZ)]
    out_shape = [jax.ShapeDtypeStruct((B, NZ), f32)] * 2

    mu, lv = pl.pallas_call(
        _kernel_body,
        grid=(nb,),
        in_specs=in_specs,
        out_specs=out_specs,
        out_shape=out_shape,
    )(v_types.astype(jnp.int32), v_pos.astype(jnp.int32),
      adj2.astype(jnp.int32), feats2.astype(f32),
      wih.astype(jnp.bfloat16), whh.astype(jnp.bfloat16), bih, bhh,
      wgm.astype(jnp.bfloat16), wgmp.astype(jnp.bfloat16), bgm,
      dfw1.astype(jnp.bfloat16), dfb1, dfw2.astype(jnp.bfloat16), dfb2,
      fc1h.astype(jnp.bfloat16), fc1d.astype(jnp.bfloat16), fc1b,
      fc2h.astype(jnp.bfloat16), fc2d.astype(jnp.bfloat16), fc2b)
    return mu, lv


# speedup vs baseline: 1.0869x; 1.0869x over previous
import jax
import jax.numpy as jnp
from jax.experimental import pallas as pl

MAX_N = 8
NVT = 26
MAX_POS = 9
HS = 301
HSP = 384
EMB = 16
FEAT_EMB = 8
NZ = 56
XDIM = NVT + MAX_POS
B = 4096
BB = 512


def _sig(x):
    return 0.5 * jnp.tanh(0.5 * x) + 0.5


def _dot(a, b):
    return jax.lax.dot_general(a.astype(jnp.bfloat16), b, (((1,), (1,)), ((), ())),
                               preferred_element_type=jnp.float32)


def _gru(gi, gh, hin):
    r = _sig(gi[:, 0:HSP] + gh[:, 0:HSP])
    z = _sig(gi[:, HSP:2 * HSP] + gh[:, HSP:2 * HSP])
    n = jnp.tanh(gi[:, 2 * HSP:3 * HSP] + r * gh[:, 2 * HSP:3 * HSP])
    return (1.0 - z) * n + z * hin


def _kernel_body(vt_ref, vp_ref, adj_ref, feats_ref,
                 wih_ref, whh_ref, bih_ref, bhh_ref,
                 wgm_ref, wgmp_ref, bgm_ref,
                 dfw1_ref, dfb1_ref, dfw2_ref, dfb2_ref,
                 fc1h_ref, fc1d_ref, fc1b_ref,
                 fc2h_ref, fc2d_ref, fc2b_ref,
                 mu_ref, lv_ref):
    vt = vt_ref[...]
    vp = vp_ref[...]
    adjf = adj_ref[...].astype(jnp.float32)
    bih = bih_ref[...]
    bhh = bhh_ref[...]
    bgm = bgm_ref[...]

    iota_x = jax.lax.broadcasted_iota(jnp.int32, (BB, XDIM), 1)
    iota_p = jax.lax.broadcasted_iota(jnp.int32, (BB, MAX_POS), 1)

    def onehot_x(v):
        t = vt[:, v:v + 1]
        p = vp[:, v:v + 1] + NVT
        return ((iota_x == t) | (iota_x == p)).astype(jnp.float32)

    def onehot_p(v):
        return (iota_p == vp[:, v:v + 1]).astype(jnp.float32)

    def gi_of(v):
        return _dot(onehot_x(v), wih_ref[...]) + bih

    h = _gru(gi_of(0), jnp.broadcast_to(bhh, (BB, 3 * HSP)), jnp.zeros((BB, HSP), jnp.float32))

    gated = []
    for v in range(1, MAX_N):
        u = v - 1
        gm = _dot(h, wgm_ref[...]) + _dot(onehot_p(u), wgmp_ref[...]) + bgm
        gated.append(_sig(gm[:, 0:HSP]) * gm[:, HSP:2 * HSP])
        hagg = jnp.zeros((BB, HSP), jnp.float32)
        for u2 in range(v):
            m = adjf[:, u2 * MAX_N + v:u2 * MAX_N + v + 1]
            hagg = hagg + m * gated[u2]
        gh = _dot(hagg, whh_ref[...]) + bhh
        h = _gru(gi_of(v), gh, hagg)

    iota_d = jax.lax.broadcasted_iota(jnp.int32, (BB, 3 * MAX_POS), 1) // 3
    df = jnp.zeros((BB, 3 * MAX_POS), jnp.float32)
    for v in range(MAX_N):
        newv = feats_ref[:, 27 * v:27 * (v + 1)]
        df = jnp.where(iota_d == vp[:, v:v + 1], newv, df)

    hd1 = jnp.maximum(_dot(df, dfw1_ref[...]) + dfb1_ref[...], 0.0)
    hd = _dot(hd1, dfw2_ref[...]) + dfb2_ref[...]

    mu_ref[...] = _dot(h, fc1h_ref[...]) + _dot(hd, fc1d_ref[...]) + fc1b_ref[...]
    lv_ref[...] = _dot(h, fc2h_ref[...]) + _dot(hd, fc2d_ref[...]) + fc2b_ref[...]


def _chunk_pad(w, nc):
    k = w.shape[1]
    return jnp.pad(w.reshape(nc, HS, k), ((0, 0), (0, HSP - HS), (0, 0))).reshape(nc * HSP, k)


@jax.jit
def kernel(v_types, v_pos, adj, feats, W_ih, W_hh, b_ih, b_hh, gate_W, gate_b,
           map_W, df_W1, df_b1, df_W2, df_b2, fc1_W, fc1_b, fc2_W, fc2_b):
    f32 = jnp.float32
    bf16 = jnp.bfloat16
    wih = _chunk_pad(W_ih, 3)
    whh = jnp.pad(W_hh.reshape(3, HS, HS),
                  ((0, 0), (0, HSP - HS), (0, HSP - HS))).reshape(3 * HSP, HSP)
    bih = _chunk_pad(b_ih[:, None], 3).reshape(1, 3 * HSP)
    bhh = _chunk_pad(b_hh[:, None], 3).reshape(1, 3 * HSP)
    gm_w = jnp.stack([gate_W, map_W])
    gm_w = jnp.pad(gm_w, ((0, 0), (0, HSP - HS), (0, 0)))
    wgm = jnp.pad(gm_w[:, :, :HS], ((0, 0), (0, 0), (0, HSP - HS))).reshape(2 * HSP, HSP)
    wgmp = gm_w[:, :, HS:].reshape(2 * HSP, MAX_POS)
    bgm = jnp.pad(gate_b[None, :], ((0, 0), (0, 2 * HSP - HS)))

    fc1h = jnp.pad(fc1_W[:, :HS], ((0, 0), (0, HSP - HS)))
    fc1d = fc1_W[:, HS:]
    fc2h = jnp.pad(fc2_W[:, :HS], ((0, 0), (0, HSP - HS)))
    fc2d = fc2_W[:, HS:]

    adj2 = adj.reshape(B, MAX_N * MAX_N)
    feats2 = jnp.tile(feats, (1, 1, MAX_POS)).reshape(B, MAX_N * 3 * MAX_POS)

    nb = B // BB
    data_spec = lambda cols: pl.BlockSpec((BB, cols), lambda i: (i, 0))
    w_spec = lambda r, c: pl.BlockSpec((r, c), lambda i: (0, 0))

    in_specs = [
        data_spec(MAX_N), data_spec(MAX_N), data_spec(MAX_N * MAX_N),
        data_spec(MAX_N * 3 * MAX_POS),
        w_spec(3 * HSP, XDIM), w_spec(3 * HSP, HSP),
        w_spec(1, 3 * HSP), w_spec(1, 3 * HSP),
        w_spec(2 * HSP, HSP), w_spec(2 * HSP, MAX_POS), w_spec(1, 2 * HSP),
        w_spec(EMB, 3 * MAX_POS), w_spec(1, EMB),
        w_spec(FEAT_EMB, EMB), w_spec(1, FEAT_EMB),
        w_spec(NZ, HSP), w_spec(NZ, FEAT_EMB), w_spec(1, NZ),
        w_spec(NZ, HSP), w_spec(NZ, FEAT_EMB), w_spec(1, NZ),
    ]
    out_specs = [data_spec(NZ), data_spec(NZ)]
    out_shape = [jax.ShapeDtypeStruct((B, NZ), f32)] * 2

    mu, lv = pl.pallas_call(
        _kernel_body,
        grid=(nb,),
        in_specs=in_specs,
        out_specs=out_specs,
        out_shape=out_shape,
    )(v_types.astype(jnp.int32), v_pos.astype(jnp.int32),
      adj2.astype(jnp.int32), feats2.astype(f32),
      wih.astype(bf16), whh.astype(bf16),
      bih.astype(f32), bhh.astype(f32),
      wgm.astype(bf16), wgmp.astype(bf16), bgm.astype(f32),
      df_W1.astype(bf16), df_b1[None, :].astype(f32),
      df_W2.astype(bf16), df_b2[None, :].astype(f32),
      fc1h.astype(bf16), fc1d.astype(bf16), fc1_b[None, :].astype(f32),
      fc2h.astype(bf16), fc2d.astype(bf16), fc2_b[None, :].astype(f32))
    return mu, lv


# speedup vs baseline: 1.1423x; 1.0510x over previous
import jax
import jax.numpy as jnp
from jax.experimental import pallas as pl

MAX_N = 8
NVT = 26
MAX_POS = 9
HS = 301
HSP = 384
VS = HS + MAX_POS
EMB = 16
FEAT_EMB = 8
NZ = 56
XDIM = NVT + MAX_POS
B = 4096
BB = 512


def _sig(x):
    return 0.5 * jnp.tanh(0.5 * x) + 0.5


def _dot(a, b):
    return jax.lax.dot_general(a.astype(jnp.bfloat16), b, (((1,), (1,)), ((), ())),
                               preferred_element_type=jnp.float32)


def _prep_body(wih_i, whh_i, bih_i, bhh_i, gw_i, gb_i, mw_i, fc1_i, fc2_i,
               dfw1_i, dfw2_i,
               wih_o, whh_o, bhh_o, wgm_o, wgmp_o,
               fc1h_o, fc1d_o, fc2h_o, fc2d_o, dfw1_o, dfw2_o):
    bf = jnp.bfloat16
    wih_o[...] = jnp.zeros(wih_o.shape, bf)
    whh_o[...] = jnp.zeros(whh_o.shape, bf)
    bhh_o[...] = jnp.zeros(bhh_o.shape, jnp.float32)
    wgm_o[...] = jnp.zeros(wgm_o.shape, bf)
    wgmp_o[...] = jnp.zeros(wgmp_o.shape, bf)
    fc1h_o[...] = jnp.zeros(fc1h_o.shape, bf)
    fc2h_o[...] = jnp.zeros(fc2h_o.shape, bf)
    for c in range(3):
        wih_o[c * HSP:c * HSP + HS, 0:XDIM] = wih_i[c * HS:(c + 1) * HS, :].astype(bf)
        wih_o[c * HSP:c * HSP + HS, XDIM:XDIM + 1] = bih_i[c * HS:(c + 1) * HS, :].astype(bf)
        whh_o[c * HSP:c * HSP + HS, 0:HS] = whh_i[c * HS:(c + 1) * HS, :].astype(bf)
        bhh_o[0:1, c * HSP:c * HSP + HS] = bhh_i[0:1, c * HS:(c + 1) * HS]
    wgm_o[0:HS, 0:HS] = gw_i[:, 0:HS].astype(bf)
    wgm_o[HSP:HSP + HS, 0:HS] = mw_i[:, 0:HS].astype(bf)
    wgmp_o[0:HS, 0:MAX_POS] = gw_i[:, HS:VS].astype(bf)
    wgmp_o[HSP:HSP + HS, 0:MAX_POS] = mw_i[:, HS:VS].astype(bf)
    wgmp_o[0:HS, MAX_POS:MAX_POS + 1] = gb_i[...].astype(bf)
    fc1h_o[:, 0:HS] = fc1_i[:, 0:HS].astype(bf)
    fc1d_o[...] = fc1_i[:, HS:VS - 1].astype(bf)
    fc2h_o[:, 0:HS] = fc2_i[:, 0:HS].astype(bf)
    fc2d_o[...] = fc2_i[:, HS:VS - 1].astype(bf)
    dfw1_o[...] = dfw1_i[...].astype(bf)
    dfw2_o[...] = dfw2_i[...].astype(bf)


def _kernel_body(vt_ref, vp_ref, adj_ref, feats_ref,
                 wih_ref, whh_ref, bhh_ref, wgm_ref, wgmp_ref,
                 dfw1_ref, dfb1_ref, dfw2_ref, dfb2_ref,
                 fc1h_ref, fc1d_ref, fc1b_ref,
                 fc2h_ref, fc2d_ref, fc2b_ref,
                 mu_ref, lv_ref):
    vt = vt_ref[...]
    vp = vp_ref[...]
    adjf = adj_ref[...].astype(jnp.float32)
    bhh = bhh_ref[...]

    iota_x = jax.lax.broadcasted_iota(jnp.int32, (BB, XDIM + 1), 1)
    iota_p = jax.lax.broadcasted_iota(jnp.int32, (BB, MAX_POS + 1), 1)

    def onehot_x(v):
        t = vt[:, v:v + 1]
        p = vp[:, v:v + 1] + NVT
        return ((iota_x == t) | (iota_x == p) | (iota_x == XDIM)).astype(jnp.float32)

    def onehot_p(v):
        return ((iota_p == vp[:, v:v + 1]) | (iota_p == MAX_POS)).astype(jnp.float32)

    def gi_of(v):
        return _dot(onehot_x(v), wih_ref[...])

    def gru(gi, gh, hin):
        rz = _sig(gi[:, 0:2 * HSP] + gh[:, 0:2 * HSP])
        r = rz[:, 0:HSP]
        z = rz[:, HSP:2 * HSP]
        n = jnp.tanh(gi[:, 2 * HSP:3 * HSP] + r * gh[:, 2 * HSP:3 * HSP])
        return n + z * (hin - n)

    h = gru(gi_of(0), jnp.broadcast_to(bhh, (BB, 3 * HSP)), jnp.zeros((BB, HSP), jnp.float32))

    gated = []
    for v in range(1, MAX_N):
        u = v - 1
        gm = _dot(h, wgm_ref[...]) + _dot(onehot_p(u), wgmp_ref[...])
        gated.append(_sig(gm[:, 0:HSP]) * gm[:, HSP:2 * HSP])
        hagg = jnp.zeros((BB, HSP), jnp.float32)
        for u2 in range(v):
            m = adjf[:, u2 * MAX_N + v:u2 * MAX_N + v + 1]
            hagg = hagg + m * gated[u2]
        gh = _dot(hagg, whh_ref[...]) + bhh
        h = gru(gi_of(v), gh, hagg)

    iota_d = jax.lax.broadcasted_iota(jnp.int32, (BB, 3 * MAX_POS), 1) // 3
    df = jnp.zeros((BB, 3 * MAX_POS), jnp.float32)
    for v in range(MAX_N):
        newv = feats_ref[:, 27 * v:27 * (v + 1)]
        df = jnp.where(iota_d == vp[:, v:v + 1], newv, df)

    hd1 = jnp.maximum(_dot(df, dfw1_ref[...]) + dfb1_ref[...], 0.0)
    hd = _dot(hd1, dfw2_ref[...]) + dfb2_ref[...]

    mu_ref[...] = _dot(h, fc1h_ref[...]) + _dot(hd, fc1d_ref[...]) + fc1b_ref[...]
    lv_ref[...] = _dot(h, fc2h_ref[...]) + _dot(hd, fc2d_ref[...]) + fc2b_ref[...]


@jax.jit
def kernel(v_types, v_pos, adj, feats, W_ih, W_hh, b_ih, b_hh, gate_W, gate_b,
           map_W, df_W1, df_b1, df_W2, df_b2, fc1_W, fc1_b, fc2_W, fc2_b):
    f32 = jnp.float32
    bf16 = jnp.bfloat16

    prep_out_shape = [
        jax.ShapeDtypeStruct((3 * HSP, XDIM + 1), bf16),
        jax.ShapeDtypeStruct((3 * HSP, HSP), bf16),
        jax.ShapeDtypeStruct((1, 3 * HSP), f32),
        jax.ShapeDtypeStruct((2 * HSP, HSP), bf16),
        jax.ShapeDtypeStruct((2 * HSP, MAX_POS + 1), bf16),
        jax.ShapeDtypeStruct((NZ, HSP), bf16),
        jax.ShapeDtypeStruct((NZ, FEAT_EMB), bf16),
        jax.ShapeDtypeStruct((NZ, HSP), bf16),
        jax.ShapeDtypeStruct((NZ, FEAT_EMB), bf16),
        jax.ShapeDtypeStruct((EMB, 3 * MAX_POS), bf16),
        jax.ShapeDtypeStruct((FEAT_EMB, EMB), bf16),
    ]
    (wih, whh, bhh, wgm, wgmp, fc1h, fc1d, fc2h, fc2d, dfw1, dfw2) = pl.pallas_call(
        _prep_body, out_shape=prep_out_shape,
    )(W_ih.astype(f32), W_hh.astype(f32), b_ih[:, None].astype(f32),
      b_hh[None, :].astype(f32), gate_W.astype(f32), gate_b[:, None].astype(f32),
      map_W.astype(f32), fc1_W.astype(f32), fc2_W.astype(f32),
      df_W1.astype(f32), df_W2.astype(f32))

    adj2 = adj.reshape(B, MAX_N * MAX_N)
    feats2 = jnp.tile(feats, (1, 1, MAX_POS)).reshape(B, MAX_N * 3 * MAX_POS)

    nb = B // BB
    data_spec = lambda cols: pl.BlockSpec((BB, cols), lambda i: (i, 0))
    w_spec = lambda r, c: pl.BlockSpec((r, c), lambda i: (0, 0))

    in_specs = [
        data_spec(MAX_N), data_spec(MAX_N), data_spec(MAX_N * MAX_N),
        data_spec(MAX_N * 3 * MAX_POS),
        w_spec(3 * HSP, XDIM + 1), w_spec(3 * HSP, HSP), w_spec(1, 3 * HSP),
        w_spec(2 * HSP, HSP), w_spec(2 * HSP, MAX_POS + 1),
        w_spec(EMB, 3 * MAX_POS), w_spec(1, EMB),
        w_spec(FEAT_EMB, EMB), w_spec(1, FEAT_EMB),
        w_spec(NZ, HSP), w_spec(NZ, FEAT_EMB), w_spec(1, NZ),
        w_spec(NZ, HSP), w_spec(NZ, FEAT_EMB), w_spec(1, NZ),
    ]
    out_specs = [data_spec(NZ), data_spec(NZ)]
    out_shape = [jax.ShapeDtypeStruct((B, NZ), f32)] * 2

    mu, lv = pl.pallas_call(
        _kernel_body,
        grid=(nb,),
        in_specs=in_specs,
        out_specs=out_specs,
        out_shape=out_shape,
    )(v_types.astype(jnp.int32), v_pos.astype(jnp.int32),
      adj2.astype(jnp.int32), feats2.astype(f32),
      wih, whh, bhh, wgm, wgmp,
      dfw1, df_b1[None, :].astype(f32), dfw2, df_b2[None, :].astype(f32),
      fc1h, fc1d, fc1_b[None, :].astype(f32),
      fc2h, fc2d, fc2_b[None, :].astype(f32))
    return mu, lv


# speedup vs baseline: 1.1592x; 1.0148x over previous
import jax
import jax.numpy as jnp
from jax.experimental import pallas as pl

MAX_N = 8
NVT = 26
MAX_POS = 9
HS = 301
HSP = 384
VS = HS + MAX_POS
EMB = 16
FEAT_EMB = 8
NZ = 56
XDIM = NVT + MAX_POS
B = 4096
BB = 1024


def _sig(x):
    return 0.5 * jnp.tanh(0.5 * x) + 0.5


def _dot(a, b):
    return jax.lax.dot_general(a.astype(jnp.bfloat16), b, (((1,), (1,)), ((), ())),
                               preferred_element_type=jnp.float32)


def _prep_body(wih_i, whh_i, bih_i, bhh_i, gw_i, gb_i, mw_i, fc1_i, fc2_i,
               dfw1_i, dfw2_i,
               wih_o, whh_o, bhh_o, wgm_o, wgmp_o,
               fc1h_o, fc1d_o, fc2h_o, fc2d_o, dfw1_o, dfw2_o):
    bf = jnp.bfloat16
    wih_o[...] = jnp.zeros(wih_o.shape, bf)
    whh_o[...] = jnp.zeros(whh_o.shape, bf)
    bhh_o[...] = jnp.zeros(bhh_o.shape, jnp.float32)
    wgm_o[...] = jnp.zeros(wgm_o.shape, bf)
    wgmp_o[...] = jnp.zeros(wgmp_o.shape, bf)
    fc1h_o[...] = jnp.zeros(fc1h_o.shape, bf)
    fc2h_o[...] = jnp.zeros(fc2h_o.shape, bf)
    for c in range(3):
        wih_o[c * HSP:c * HSP + HS, 0:XDIM] = wih_i[c * HS:(c + 1) * HS, :].astype(bf)
        wih_o[c * HSP:c * HSP + HS, XDIM:XDIM + 1] = bih_i[c * HS:(c + 1) * HS, :].astype(bf)
        whh_o[c * HSP:c * HSP + HS, 0:HS] = whh_i[c * HS:(c + 1) * HS, :].astype(bf)
        bhh_o[0:1, c * HSP:c * HSP + HS] = bhh_i[0:1, c * HS:(c + 1) * HS]
    wgm_o[0:HS, 0:HS] = gw_i[:, 0:HS].astype(bf)
    wgm_o[HSP:HSP + HS, 0:HS] = mw_i[:, 0:HS].astype(bf)
    wgmp_o[0:HS, 0:MAX_POS] = gw_i[:, HS:VS].astype(bf)
    wgmp_o[HSP:HSP + HS, 0:MAX_POS] = mw_i[:, HS:VS].astype(bf)
    wgmp_o[0:HS, MAX_POS:MAX_POS + 1] = gb_i[...].astype(bf)
    fc1h_o[:, 0:HS] = fc1_i[:, 0:HS].astype(bf)
    fc1d_o[...] = fc1_i[:, HS:VS - 1].astype(bf)
    fc2h_o[:, 0:HS] = fc2_i[:, 0:HS].astype(bf)
    fc2d_o[...] = fc2_i[:, HS:VS - 1].astype(bf)
    dfw1_o[...] = dfw1_i[...].astype(bf)
    dfw2_o[...] = dfw2_i[...].astype(bf)


def _kernel_body(vt_ref, vp_ref, adj_ref, feats_ref,
                 wih_ref, whh_ref, bhh_ref, wgm_ref, wgmp_ref,
                 dfw1_ref, dfb1_ref, dfw2_ref, dfb2_ref,
                 fc1h_ref, fc1d_ref, fc1b_ref,
                 fc2h_ref, fc2d_ref, fc2b_ref,
                 mu_ref, lv_ref):
    vt = vt_ref[...]
    vp = vp_ref[...]
    adjf = adj_ref[...].astype(jnp.float32)
    bhh = bhh_ref[...]

    iota_x = jax.lax.broadcasted_iota(jnp.int32, (BB, XDIM + 1), 1)
    iota_p = jax.lax.broadcasted_iota(jnp.int32, (BB, MAX_POS + 1), 1)

    def onehot_x(v):
        t = vt[:, v:v + 1]
        p = vp[:, v:v + 1] + NVT
        return ((iota_x == t) | (iota_x == p) | (iota_x == XDIM)).astype(jnp.float32)

    def onehot_p(v):
        return ((iota_p == vp[:, v:v + 1]) | (iota_p == MAX_POS)).astype(jnp.float32)

    def gi_of(v):
        return _dot(onehot_x(v), wih_ref[...])

    def gru(gi, gh, hin):
        rz = _sig(gi[:, 0:2 * HSP] + gh[:, 0:2 * HSP])
        r = rz[:, 0:HSP]
        z = rz[:, HSP:2 * HSP]
        n = jnp.tanh(gi[:, 2 * HSP:3 * HSP] + r * gh[:, 2 * HSP:3 * HSP])
        return n + z * (hin - n)

    h = gru(gi_of(0), jnp.broadcast_to(bhh, (BB, 3 * HSP)), jnp.zeros((BB, HSP), jnp.float32))

    gated = []
    for v in range(1, MAX_N):
        u = v - 1
        gm = _dot(h, wgm_ref[...]) + _dot(onehot_p(u), wgmp_ref[...])
        gated.append(_sig(gm[:, 0:HSP]) * gm[:, HSP:2 * HSP])
        hagg = jnp.zeros((BB, HSP), jnp.float32)
        for u2 in range(v):
            m = adjf[:, u2 * MAX_N + v:u2 * MAX_N + v + 1]
            hagg = hagg + m * gated[u2]
        gh = _dot(hagg, whh_ref[...]) + bhh
        h = gru(gi_of(v), gh, hagg)

    iota_d = jax.lax.broadcasted_iota(jnp.int32, (BB, 3 * MAX_POS), 1) // 3
    df = jnp.zeros((BB, 3 * MAX_POS), jnp.float32)
    for v in range(MAX_N):
        newv = feats_ref[:, 27 * v:27 * (v + 1)]
        df = jnp.where(iota_d == vp[:, v:v + 1], newv, df)

    hd1 = jnp.maximum(_dot(df, dfw1_ref[...]) + dfb1_ref[...], 0.0)
    hd = _dot(hd1, dfw2_ref[...]) + dfb2_ref[...]

    mu_ref[...] = _dot(h, fc1h_ref[...]) + _dot(hd, fc1d_ref[...]) + fc1b_ref[...]
    lv_ref[...] = _dot(h, fc2h_ref[...]) + _dot(hd, fc2d_ref[...]) + fc2b_ref[...]


@jax.jit
def kernel(v_types, v_pos, adj, feats, W_ih, W_hh, b_ih, b_hh, gate_W, gate_b,
           map_W, df_W1, df_b1, df_W2, df_b2, fc1_W, fc1_b, fc2_W, fc2_b):
    f32 = jnp.float32
    bf16 = jnp.bfloat16

    prep_out_shape = [
        jax.ShapeDtypeStruct((3 * HSP, XDIM + 1), bf16),
        jax.ShapeDtypeStruct((3 * HSP, HSP), bf16),
        jax.ShapeDtypeStruct((1, 3 * HSP), f32),
        jax.ShapeDtypeStruct((2 * HSP, HSP), bf16),
        jax.ShapeDtypeStruct((2 * HSP, MAX_POS + 1), bf16),
        jax.ShapeDtypeStruct((NZ, HSP), bf16),
        jax.ShapeDtypeStruct((NZ, FEAT_EMB), bf16),
        jax.ShapeDtypeStruct((NZ, HSP), bf16),
        jax.ShapeDtypeStruct((NZ, FEAT_EMB), bf16),
        jax.ShapeDtypeStruct((EMB, 3 * MAX_POS), bf16),
        jax.ShapeDtypeStruct((FEAT_EMB, EMB), bf16),
    ]
    (wih, whh, bhh, wgm, wgmp, fc1h, fc1d, fc2h, fc2d, dfw1, dfw2) = pl.pallas_call(
        _prep_body, out_shape=prep_out_shape,
    )(W_ih.astype(f32), W_hh.astype(f32), b_ih[:, None].astype(f32),
      b_hh[None, :].astype(f32), gate_W.astype(f32), gate_b[:, None].astype(f32),
      map_W.astype(f32), fc1_W.astype(f32), fc2_W.astype(f32),
      df_W1.astype(f32), df_W2.astype(f32))

    adj2 = adj.reshape(B, MAX_N * MAX_N)
    feats2 = jnp.tile(feats, (1, 1, MAX_POS)).reshape(B, MAX_N * 3 * MAX_POS)

    nb = B // BB
    data_spec = lambda cols: pl.BlockSpec((BB, cols), lambda i: (i, 0))
    w_spec = lambda r, c: pl.BlockSpec((r, c), lambda i: (0, 0))

    in_specs = [
        data_spec(MAX_N), data_spec(MAX_N), data_spec(MAX_N * MAX_N),
        data_spec(MAX_N * 3 * MAX_POS),
        w_spec(3 * HSP, XDIM + 1), w_spec(3 * HSP, HSP), w_spec(1, 3 * HSP),
        w_spec(2 * HSP, HSP), w_spec(2 * HSP, MAX_POS + 1),
        w_spec(EMB, 3 * MAX_POS), w_spec(1, EMB),
        w_spec(FEAT_EMB, EMB), w_spec(1, FEAT_EMB),
        w_spec(NZ, HSP), w_spec(NZ, FEAT_EMB), w_spec(1, NZ),
        w_spec(NZ, HSP), w_spec(NZ, FEAT_EMB), w_spec(1, NZ),
    ]
    out_specs = [data_spec(NZ), data_spec(NZ)]
    out_shape = [jax.ShapeDtypeStruct((B, NZ), f32)] * 2

    mu, lv = pl.pallas_call(
        _kernel_body,
        grid=(nb,),
        in_specs=in_specs,
        out_specs=out_specs,
        out_shape=out_shape,
    )(v_types.astype(jnp.int32), v_pos.astype(jnp.int32),
      adj2.astype(jnp.int32), feats2.astype(f32),
      wih, whh, bhh, wgm, wgmp,
      dfw1, df_b1[None, :].astype(f32), dfw2, df_b2[None, :].astype(f32),
      fc1h, fc1d, fc1_b[None, :].astype(f32),
      fc2h, fc2d, fc2_b[None, :].astype(f32))
    return mu, lv


# speedup vs baseline: 1.1641x; 1.0042x over previous
import jax
import jax.numpy as jnp
from jax.experimental import pallas as pl

MAX_N = 8
NVT = 26
MAX_POS = 9
HS = 301
HSP = 384
VS = HS + MAX_POS
EMB = 16
FEAT_EMB = 8
NZ = 56
XDIM = NVT + MAX_POS
B = 4096
BB = 1024


def _sig(x):
    return 0.5 * jnp.tanh(0.5 * x) + 0.5


def _dot(a, b):
    return jax.lax.dot_general(a.astype(jnp.bfloat16), b, (((1,), (1,)), ((), ())),
                               preferred_element_type=jnp.float32)


def _prep_body(wih_i, whh_i, bih_i, bhh_i, gw_i, gb_i, mw_i, fc1_i, fc2_i,
               dfw1_i, dfw2_i,
               wih_o, whh_o, bhh_o, wgm_o, wgmp_o,
               fc1h_o, fc1d_o, fc2h_o, fc2d_o, dfw1_o, dfw2_o):
    bf = jnp.bfloat16
    wih_o[...] = jnp.zeros(wih_o.shape, bf)
    whh_o[...] = jnp.zeros(whh_o.shape, bf)
    bhh_o[...] = jnp.zeros(bhh_o.shape, jnp.float32)
    wgm_o[...] = jnp.zeros(wgm_o.shape, bf)
    wgmp_o[...] = jnp.zeros(wgmp_o.shape, bf)
    fc1h_o[...] = jnp.zeros(fc1h_o.shape, bf)
    fc2h_o[...] = jnp.zeros(fc2h_o.shape, bf)
    for c in range(3):
        wih_o[c * HSP:c * HSP + HS, 0:XDIM] = wih_i[c * HS:(c + 1) * HS, :].astype(bf)
        wih_o[c * HSP:c * HSP + HS, XDIM:XDIM + 1] = bih_i[c * HS:(c + 1) * HS, :].astype(bf)
        whh_o[c * HSP:c * HSP + HS, 0:HS] = whh_i[c * HS:(c + 1) * HS, :].astype(bf)
        bhh_o[0:1, c * HSP:c * HSP + HS] = bhh_i[0:1, c * HS:(c + 1) * HS]
    wgm_o[0:HS, 0:HS] = gw_i[:, 0:HS].astype(bf)
    wgm_o[HSP:HSP + HS, 0:HS] = mw_i[:, 0:HS].astype(bf)
    wgmp_o[0:HS, 0:MAX_POS] = gw_i[:, HS:VS].astype(bf)
    wgmp_o[HSP:HSP + HS, 0:MAX_POS] = mw_i[:, HS:VS].astype(bf)
    wgmp_o[0:HS, MAX_POS:MAX_POS + 1] = gb_i[...].astype(bf)
    fc1h_o[:, 0:HS] = fc1_i[:, 0:HS].astype(bf)
    fc1d_o[...] = fc1_i[:, HS:VS - 1].astype(bf)
    fc2h_o[:, 0:HS] = fc2_i[:, 0:HS].astype(bf)
    fc2d_o[...] = fc2_i[:, HS:VS - 1].astype(bf)
    dfw1_o[...] = dfw1_i[...].astype(bf)
    dfw2_o[...] = dfw2_i[...].astype(bf)


def _kernel_body(vt_ref, vp_ref, adj_ref, feats_ref,
                 wih_ref, whh_ref, bhh_ref, wgm_ref, wgmp_ref,
                 dfw1_ref, dfb1_ref, dfw2_ref, dfb2_ref,
                 fc1h_ref, fc1d_ref, fc1b_ref,
                 fc2h_ref, fc2d_ref, fc2b_ref,
                 mu_ref, lv_ref):
    HB = BB // 2
    bhh = bhh_ref[...]
    vt, vp, adjf = [], [], []
    for s in range(2):
        r0, r1 = s * HB, (s + 1) * HB
        vt.append(vt_ref[r0:r1, :])
        vp.append(vp_ref[r0:r1, :])
        adjf.append(adj_ref[r0:r1, :].astype(jnp.float32))

    iota_x = jax.lax.broadcasted_iota(jnp.int32, (HB, XDIM + 1), 1)
    iota_p = jax.lax.broadcasted_iota(jnp.int32, (HB, MAX_POS + 1), 1)

    def onehot_x(s, v):
        t = vt[s][:, v:v + 1]
        p = vp[s][:, v:v + 1] + NVT
        return ((iota_x == t) | (iota_x == p) | (iota_x == XDIM)).astype(jnp.float32)

    def onehot_p(s, v):
        return ((iota_p == vp[s][:, v:v + 1]) | (iota_p == MAX_POS)).astype(jnp.float32)

    def gi_of(s, v):
        return _dot(onehot_x(s, v), wih_ref[...])

    def gru(gi, gh, hin):
        rz = _sig(gi[:, 0:2 * HSP] + gh[:, 0:2 * HSP])
        r = rz[:, 0:HSP]
        z = rz[:, HSP:2 * HSP]
        n = jnp.tanh(gi[:, 2 * HSP:3 * HSP] + r * gh[:, 2 * HSP:3 * HSP])
        return n + z * (hin - n)

    bhh_b = jnp.broadcast_to(bhh, (HB, 3 * HSP))
    zero_h = jnp.zeros((HB, HSP), jnp.float32)
    h = [gru(gi_of(s, 0), bhh_b, zero_h) for s in range(2)]
    gated = [[], []]
    for v in range(1, MAX_N):
        u = v - 1
        for s in range(2):
            gm = _dot(h[s], wgm_ref[...]) + _dot(onehot_p(s, u), wgmp_ref[...])
            gated[s].append(_sig(gm[:, 0:HSP]) * gm[:, HSP:2 * HSP])
            hagg = zero_h
            for u2 in range(v):
                m = adjf[s][:, u2 * MAX_N + v:u2 * MAX_N + v + 1]
                hagg = hagg + m * gated[s][u2]
            gh = _dot(hagg, whh_ref[...]) + bhh
            h[s] = gru(gi_of(s, v), gh, hagg)

    iota_d = jax.lax.broadcasted_iota(jnp.int32, (HB, 3 * MAX_POS), 1) // 3
    for s in range(2):
        r0, r1 = s * HB, (s + 1) * HB
        df = jnp.zeros((HB, 3 * MAX_POS), jnp.float32)
        for v in range(MAX_N):
            newv = feats_ref[r0:r1, 27 * v:27 * (v + 1)]
            df = jnp.where(iota_d == vp[s][:, v:v + 1], newv, df)

        hd1 = jnp.maximum(_dot(df, dfw1_ref[...]) + dfb1_ref[...], 0.0)
        hd = _dot(hd1, dfw2_ref[...]) + dfb2_ref[...]

        mu_ref[r0:r1, :] = _dot(h[s], fc1h_ref[...]) + _dot(hd, fc1d_ref[...]) + fc1b_ref[...]
        lv_ref[r0:r1, :] = _dot(h[s], fc2h_ref[...]) + _dot(hd, fc2d_ref[...]) + fc2b_ref[...]


@jax.jit
def kernel(v_types, v_pos, adj, feats, W_ih, W_hh, b_ih, b_hh, gate_W, gate_b,
           map_W, df_W1, df_b1, df_W2, df_b2, fc1_W, fc1_b, fc2_W, fc2_b):
    f32 = jnp.float32
    bf16 = jnp.bfloat16

    prep_out_shape = [
        jax.ShapeDtypeStruct((3 * HSP, XDIM + 1), bf16),
        jax.ShapeDtypeStruct((3 * HSP, HSP), bf16),
        jax.ShapeDtypeStruct((1, 3 * HSP), f32),
        jax.ShapeDtypeStruct((2 * HSP, HSP), bf16),
        jax.ShapeDtypeStruct((2 * HSP, MAX_POS + 1), bf16),
        jax.ShapeDtypeStruct((NZ, HSP), bf16),
        jax.ShapeDtypeStruct((NZ, FEAT_EMB), bf16),
        jax.ShapeDtypeStruct((NZ, HSP), bf16),
        jax.ShapeDtypeStruct((NZ, FEAT_EMB), bf16),
        jax.ShapeDtypeStruct((EMB, 3 * MAX_POS), bf16),
        jax.ShapeDtypeStruct((FEAT_EMB, EMB), bf16),
    ]
    (wih, whh, bhh, wgm, wgmp, fc1h, fc1d, fc2h, fc2d, dfw1, dfw2) = pl.pallas_call(
        _prep_body, out_shape=prep_out_shape,
    )(W_ih.astype(f32), W_hh.astype(f32), b_ih[:, None].astype(f32),
      b_hh[None, :].astype(f32), gate_W.astype(f32), gate_b[:, None].astype(f32),
      map_W.astype(f32), fc1_W.astype(f32), fc2_W.astype(f32),
      df_W1.astype(f32), df_W2.astype(f32))

    adj2 = adj.reshape(B, MAX_N * MAX_N)
    feats2 = jnp.tile(feats, (1, 1, MAX_POS)).reshape(B, MAX_N * 3 * MAX_POS)

    nb = B // BB
    data_spec = lambda cols: pl.BlockSpec((BB, cols), lambda i: (i, 0))
    w_spec = lambda r, c: pl.BlockSpec((r, c), lambda i: (0, 0))

    in_specs = [
        data_spec(MAX_N), data_spec(MAX_N), data_spec(MAX_N * MAX_N),
        data_spec(MAX_N * 3 * MAX_POS),
        w_spec(3 * HSP, XDIM + 1), w_spec(3 * HSP, HSP), w_spec(1, 3 * HSP),
        w_spec(2 * HSP, HSP), w_spec(2 * HSP, MAX_POS + 1),
        w_spec(EMB, 3 * MAX_POS), w_spec(1, EMB),
        w_spec(FEAT_EMB, EMB), w_spec(1, FEAT_EMB),
        w_spec(NZ, HSP), w_spec(NZ, FEAT_EMB), w_spec(1, NZ),
        w_spec(NZ, HSP), w_spec(NZ, FEAT_EMB), w_spec(1, NZ),
    ]
    out_specs = [data_spec(NZ), data_spec(NZ)]
    out_shape = [jax.ShapeDtypeStruct((B, NZ), f32)] * 2

    mu, lv = pl.pallas_call(
        _kernel_body,
        grid=(nb,),
        in_specs=in_specs,
        out_specs=out_specs,
        out_shape=out_shape,
    )(v_types.astype(jnp.int32), v_pos.astype(jnp.int32),
      adj2.astype(jnp.int32), feats2.astype(f32),
      wih, whh, bhh, wgm, wgmp,
      dfw1, df_b1[None, :].astype(f32), dfw2, df_b2[None, :].astype(f32),
      fc1h, fc1d, fc1_b[None, :].astype(f32),
      fc2h, fc2d, fc2_b[None, :].astype(f32))
    return mu, lv
